# Initial kernel scaffold; baseline (speedup 1.0000x reference)
import jax
import jax.numpy as jnp
from jax.experimental import pallas as pl


def kernel(x, lut):
    raise NotImplementedError("write your pallas kernel here")


# speedup vs baseline: 2.4097x; 2.4097x over previous
import functools
import math

import jax
import jax.numpy as jnp
from jax import lax
from jax.experimental import pallas as pl
from jax.experimental.pallas import tpu as pltpu
from jax.experimental.pallas import tpu_sc as plsc

NC = 2
NS = 16
NW = NC * NS
LANES = 16
CH = 128


@functools.partial(jax.jit, static_argnums=(2, 3))
def _sc_embed(lut, idx, n_rows, d):
    b_per_w = n_rows // NW
    n_ch = b_per_w // CH
    scale = jnp.float32(math.sqrt(d))
    mesh = plsc.VectorSubcoreMesh(
        core_axis_name="c", subcore_axis_name="s", num_cores=NC, num_subcores=NS
    )

    @functools.partial(
        pl.kernel,
        out_type=jax.ShapeDtypeStruct((n_rows, d), jnp.float32),
        mesh=mesh,
        scratch_types=[
            pltpu.VMEM((b_per_w,), jnp.int32),
            pltpu.VMEM((CH, d), jnp.float32),
            pltpu.SemaphoreType.DMA,
        ],
    )
    def k(lut_hbm, idx_hbm, out_hbm, idx_v, buf, sem):
        wid = lax.axis_index("s") * NC + lax.axis_index("c")
        base = wid * b_per_w
        pltpu.sync_copy(idx_hbm.at[pl.ds(base, b_per_w)], idx_v)

        def chunk(j, carry):
            idx_sl = idx_v.at[pl.ds(j * CH, CH)]
            pltpu.async_copy(lut_hbm.at[idx_sl], buf, sem).wait()

            def scale_row(i, c):
                for t in range(d // LANES):
                    sl = pl.ds(t * LANES, LANES)
                    buf[i, sl] = buf[i, sl] * scale
                return c

            lax.fori_loop(0, CH, scale_row, 0)
            pltpu.sync_copy(buf, out_hbm.at[pl.ds(base + j * CH, CH)])
            return carry

        lax.fori_loop(0, n_ch, chunk, 0)

    return k(lut, idx)


def kernel(x, lut):
    batch, hist = x.shape
    vocab, d = lut.shape
    n_rows = batch * hist
    idx = x.reshape(n_rows)
    out = _sc_embed(lut, idx, n_rows, d)
    return out.reshape(batch, hist, d)


# speedup vs baseline: 2.9295x; 1.2157x over previous
import functools
import math

import jax
import jax.numpy as jnp
from jax import lax
from jax.experimental import pallas as pl
from jax.experimental.pallas import tpu as pltpu
from jax.experimental.pallas import tpu_sc as plsc

NC = 2
NS = 16
NW = NC * NS
LANES = 16
CH = 128
NBUF = 4
PRE = 2


@functools.partial(jax.jit, static_argnums=(2, 3))
def _sc_embed(lut, idx, n_rows, d):
    b_per_w = n_rows // NW
    n = b_per_w // CH
    n_groups = (n + NBUF - 1) // NBUF
    scale = jnp.float32(math.sqrt(d))
    mesh = plsc.VectorSubcoreMesh(
        core_axis_name="c", subcore_axis_name="s", num_cores=NC, num_subcores=NS
    )

    @functools.partial(
        pl.kernel,
        out_type=jax.ShapeDtypeStruct((n_rows, d), jnp.float32),
        mesh=mesh,
        scratch_types=[
            pltpu.VMEM((b_per_w,), jnp.int32),
            pltpu.VMEM((NBUF, CH, d), jnp.float32),
        ]
        + [pltpu.SemaphoreType.DMA] * (2 * NBUF),
    )
    def k(lut_hbm, idx_hbm, out_hbm, idx_v, bufs, *sems):
        gsem = sems[:NBUF]
        ssem = sems[NBUF:]
        wid = lax.axis_index("s") * NC + lax.axis_index("c")
        base = wid * b_per_w
        pltpu.sync_copy(idx_hbm.at[pl.ds(base, b_per_w)], idx_v)

        def gather(j, b):
            return pltpu.make_async_copy(
                lut_hbm.at[idx_v.at[pl.ds(j * CH, CH)]], bufs.at[b], gsem[b]
            )

        def scatter(j, b):
            return pltpu.make_async_copy(
                bufs.at[b], out_hbm.at[pl.ds(base + j * CH, CH)], ssem[b]
            )

        for b in range(PRE):
            gather(b, b).start()

        def group(g, carry):
            for b in range(NBUF):
                j = g * NBUF + b
                jp = j + PRE
                bp = (b + PRE) % NBUF

                @pl.when((jp < n) & (j >= PRE))
                def _():
                    scatter(jp - NBUF, bp).wait()
                    gather(jp, bp).start()

                @pl.when((jp < n) & (j < PRE))
                def _():
                    gather(jp, bp).start()

                @pl.when(j < n)
                def _():
                    gather(j, b).wait()

                    @plsc.parallel_loop(0, CH, step=1, unroll=4)
                    def _(i):
                        for t in range(d // LANES):
                            sl = pl.ds(t * LANES, LANES)
                            bufs[b, i, sl] = bufs[b, i, sl] * scale

                    scatter(j, b).start()
            return carry

        lax.fori_loop(0, n_groups, group, 0)

        for j in range(max(0, n - NBUF), n):
            scatter(j, j % NBUF).wait()

    return k(lut, idx)


def kernel(x, lut):
    batch, hist = x.shape
    vocab, d = lut.shape
    n_rows = batch * hist
    idx = x.reshape(n_rows)
    out = _sc_embed(lut, idx, n_rows, d)
    return out.reshape(batch, hist, d)


# speedup vs baseline: 2.9440x; 1.0050x over previous
import functools
import math

import jax
import jax.numpy as jnp
from jax import lax
from jax.experimental import pallas as pl
from jax.experimental.pallas import tpu as pltpu
from jax.experimental.pallas import tpu_sc as plsc

NC = 2
NS = 16
NW = NC * NS
LANES = 16
CH = 128
NBUF = 6
PRE = 3


@functools.partial(jax.jit, static_argnums=(2, 3))
def _sc_embed(lut, idx, n_rows, d):
    b_per_w = n_rows // NW
    n = b_per_w // CH
    n_groups = (n + NBUF - 1) // NBUF
    scale = jnp.float32(math.sqrt(d))
    mesh = plsc.VectorSubcoreMesh(
        core_axis_name="c", subcore_axis_name="s", num_cores=NC, num_subcores=NS
    )

    @functools.partial(
        pl.kernel,
        out_type=jax.ShapeDtypeStruct((n_rows, d), jnp.float32),
        mesh=mesh,
        scratch_types=[
            pltpu.VMEM((b_per_w,), jnp.int32),
            pltpu.VMEM((NBUF, CH, d), jnp.float32),
        ]
        + [pltpu.SemaphoreType.DMA] * (2 * NBUF),
    )
    def k(lut_hbm, idx_hbm, out_hbm, idx_v, bufs, *sems):
        gsem = sems[:NBUF]
        ssem = sems[NBUF:]
        wid = lax.axis_index("s") * NC + lax.axis_index("c")
        base = wid * b_per_w
        pltpu.sync_copy(idx_hbm.at[pl.ds(base, b_per_w)], idx_v)

        def gather(j, b):
            return pltpu.make_async_copy(
                lut_hbm.at[idx_v.at[pl.ds(j * CH, CH)]], bufs.at[b], gsem[b]
            )

        def scatter(j, b):
            return pltpu.make_async_copy(
                bufs.at[b], out_hbm.at[pl.ds(base + j * CH, CH)], ssem[b]
            )

        for b in range(PRE):
            gather(b, b).start()

        def group(g, carry):
            for b in range(NBUF):
                j = g * NBUF + b
                jp = j + PRE
                bp = (b + PRE) % NBUF

                @pl.when((jp < n) & (j >= NBUF - PRE))
                def _():
                    scatter(jp - NBUF, bp).wait()
                    gather(jp, bp).start()

                @pl.when((jp < n) & (j < NBUF - PRE))
                def _():
                    gather(jp, bp).start()

                @pl.when(j < n)
                def _():
                    gather(j, b).wait()

                    @plsc.parallel_loop(0, CH, step=1, unroll=4)
                    def _(i):
                        for t in range(d // LANES):
                            sl = pl.ds(t * LANES, LANES)
                            bufs[b, i, sl] = bufs[b, i, sl] * scale

                    scatter(j, b).start()
            return carry

        lax.fori_loop(0, n_groups, group, 0)

        for j in range(max(0, n - NBUF), n):
            scatter(j, j % NBUF).wait()

    return k(lut, idx)


def kernel(x, lut):
    batch, hist = x.shape
    vocab, d = lut.shape
    n_rows = batch * hist
    idx = x.reshape(n_rows)
    out = _sc_embed(lut, idx, n_rows, d)
    return out.reshape(batch, hist, d)


# speedup vs baseline: 5.2345x; 1.7780x over previous
import functools
import math

import jax
import jax.numpy as jnp
from jax import lax
from jax.experimental import pallas as pl
from jax.experimental.pallas import tpu as pltpu
from jax.experimental.pallas import tpu_sc as plsc

NC = 2
NS = 16
NW = NC * NS
LANES = 16
NBUF = 6
PRE = 3


@jax.jit
def _sc_embed(lut, x):
    batch, hist = x.shape
    _, d = lut.shape
    n = batch // NW
    n_groups = (n + NBUF - 1) // NBUF
    scale = jnp.float32(math.sqrt(d))
    mesh = plsc.VectorSubcoreMesh(
        core_axis_name="c", subcore_axis_name="s", num_cores=NC, num_subcores=NS
    )

    @functools.partial(
        pl.kernel,
        out_type=jax.ShapeDtypeStruct((batch, hist, d), jnp.float32),
        mesh=mesh,
        scratch_types=[
            pltpu.VMEM((n, hist), jnp.int32),
            pltpu.VMEM((NBUF, hist, d), jnp.float32),
        ]
        + [pltpu.SemaphoreType.DMA] * (2 * NBUF),
    )
    def k(lut_hbm, idx_hbm, out_hbm, idx_v, bufs, *sems):
        gsem = sems[:NBUF]
        ssem = sems[NBUF:]
        wid = lax.axis_index("s") * NC + lax.axis_index("c")
        base = wid * n
        pltpu.sync_copy(idx_hbm.at[pl.ds(base, n)], idx_v)

        def gather(j, b):
            return pltpu.make_async_copy(
                lut_hbm.at[idx_v.at[j]], bufs.at[b], gsem[b]
            )

        def scatter(j, b):
            return pltpu.make_async_copy(
                bufs.at[b], out_hbm.at[base + j], ssem[b]
            )

        for b in range(PRE):
            gather(b, b).start()

        def group(g, carry):
            for b in range(NBUF):
                j = g * NBUF + b
                jp = j + PRE
                bp = (b + PRE) % NBUF

                @pl.when((jp < n) & (j >= NBUF - PRE))
                def _():
                    scatter(jp - NBUF, bp).wait()
                    gather(jp, bp).start()

                @pl.when((jp < n) & (j < NBUF - PRE))
                def _():
                    gather(jp, bp).start()

                @pl.when(j < n)
                def _():
                    gather(j, b).wait()

                    @plsc.parallel_loop(0, hist, step=1, unroll=2)
                    def _(i):
                        for t in range(d // LANES):
                            sl = pl.ds(t * LANES, LANES)
                            bufs[b, i, sl] = bufs[b, i, sl] * scale

                    scatter(j, b).start()
            return carry

        lax.fori_loop(0, n_groups, group, 0)

        for j in range(max(0, n - NBUF), n):
            scatter(j, j % NBUF).wait()

    return k(lut, x)


def kernel(x, lut):
    return _sc_embed(lut, x)


# speedup vs baseline: 5.2505x; 1.0031x over previous
import functools
import math

import jax
import jax.numpy as jnp
from jax import lax
from jax.experimental import pallas as pl
from jax.experimental.pallas import tpu as pltpu
from jax.experimental.pallas import tpu_sc as plsc

NC = 2
NS = 16
NW = NC * NS
LANES = 16
NBUF = 6
PRE = 3


@jax.jit
def _sc_embed(lut, x):
    batch, hist = x.shape
    _, d = lut.shape
    n = batch // NW
    n_groups = (n + NBUF - 1) // NBUF
    scale = jnp.float32(math.sqrt(d))
    mesh = plsc.VectorSubcoreMesh(
        core_axis_name="c", subcore_axis_name="s", num_cores=NC, num_subcores=NS
    )

    @functools.partial(
        pl.kernel,
        out_type=jax.ShapeDtypeStruct((batch, hist, d), jnp.float32),
        mesh=mesh,
        scratch_types=[
            pltpu.VMEM((n, hist), jnp.int32),
            pltpu.VMEM((NBUF, hist, d), jnp.float32),
        ]
        + [pltpu.SemaphoreType.DMA] * (2 * NBUF),
        compiler_params=pltpu.CompilerParams(use_tc_tiling_on_sc=True),
    )
    def k(lut_hbm, idx_hbm, out_hbm, idx_v, bufs, *sems):
        gsem = sems[:NBUF]
        ssem = sems[NBUF:]
        wid = lax.axis_index("s") * NC + lax.axis_index("c")
        base = wid * n
        pltpu.sync_copy(idx_hbm.at[pl.ds(base, n)], idx_v)

        def gather(j, b):
            return pltpu.make_async_copy(
                lut_hbm.at[idx_v.at[j]], bufs.at[b], gsem[b]
            )

        def scatter(j, b):
            return pltpu.make_async_copy(
                bufs.at[b], out_hbm.at[base + j], ssem[b]
            )

        for b in range(PRE):
            gather(b, b).start()

        def group(g, carry):
            for b in range(NBUF):
                j = g * NBUF + b
                jp = j + PRE
                bp = (b + PRE) % NBUF

                @pl.when((jp < n) & (j >= NBUF - PRE))
                def _():
                    scatter(jp - NBUF, bp).wait()
                    gather(jp, bp).start()

                @pl.when((jp < n) & (j < NBUF - PRE))
                def _():
                    gather(jp, bp).start()

                @pl.when(j < n)
                def _():
                    gather(j, b).wait()

                    @plsc.parallel_loop(0, hist, step=1, unroll=2)
                    def _(i):
                        for t in range(d // LANES):
                            sl = pl.ds(t * LANES, LANES)
                            bufs[b, i, sl] = bufs[b, i, sl] * scale

                    scatter(j, b).start()
            return carry

        lax.fori_loop(0, n_groups, group, 0)

        for j in range(max(0, n - NBUF), n):
            scatter(j, j % NBUF).wait()

    return k(lut, x)


def kernel(x, lut):
    return _sc_embed(lut, x)


# speedup vs baseline: 9.4059x; 1.7914x over previous
import functools
import math

import jax
import jax.numpy as jnp
from jax import lax
from jax.experimental import pallas as pl
from jax.experimental.pallas import tpu as pltpu
from jax.experimental.pallas import tpu_sc as plsc

NC = 2
NS = 16
NW = NC * NS
LANES = 16
NBUF = 7
PRE = 3


@jax.jit
def _sc_embed(lut, xt):
    hist, batch = xt.shape
    _, d = lut.shape
    bcols = batch // NW
    n = hist
    n_groups = (n + NBUF - 1) // NBUF
    scale = float(math.sqrt(d))
    mesh = plsc.VectorSubcoreMesh(
        core_axis_name="c", subcore_axis_name="s", num_cores=NC, num_subcores=NS
    )

    @functools.partial(
        pl.kernel,
        out_type=jax.ShapeDtypeStruct((hist, batch, d), jnp.float32),
        mesh=mesh,
        scratch_types=[
            pltpu.VMEM((hist, bcols), jnp.int32),
            pltpu.VMEM((NBUF, bcols, d), jnp.float32),
        ]
        + [pltpu.SemaphoreType.DMA] * (2 * NBUF),
    )
    def k(lut_hbm, idx_hbm, out_hbm, idx_v, bufs, *sems):
        gsem = sems[:NBUF]
        ssem = sems[NBUF:]
        wid = lax.axis_index("s") * NC + lax.axis_index("c")
        base = wid * bcols
        pltpu.sync_copy(idx_hbm.at[:, pl.ds(base, bcols)], idx_v)

        def gather(j, b):
            return pltpu.make_async_copy(
                lut_hbm.at[idx_v.at[j]], bufs.at[b], gsem[b]
            )

        def scatter(j, b):
            return pltpu.make_async_copy(
                bufs.at[b], out_hbm.at[j, pl.ds(base, bcols)], ssem[b]
            )

        for b in range(PRE):
            gather(b, b).start()

        def group(g, carry):
            for b in range(NBUF):
                j = g * NBUF + b
                jp = j + PRE
                bp = (b + PRE) % NBUF

                @pl.when((jp < n) & (j >= NBUF - PRE))
                def _():
                    scatter(jp - NBUF, bp).wait()
                    gather(jp, bp).start()

                @pl.when((jp < n) & (j < NBUF - PRE))
                def _():
                    gather(jp, bp).start()

                @pl.when(j < n)
                def _():
                    gather(j, b).wait()

                    @plsc.parallel_loop(0, bcols, step=1, unroll=2)
                    def _(i):
                        for t in range(d // LANES):
                            sl = pl.ds(t * LANES, LANES)
                            bufs[b, i, sl] = bufs[b, i, sl] * scale

                    scatter(j, b).start()
            return carry

        lax.fori_loop(0, n_groups, group, 0)

        for j in range(max(0, n - NBUF), n):
            scatter(j, j % NBUF).wait()

    return k(lut, xt)


def kernel(x, lut):
    out_t = _sc_embed(lut, x.T)
    return out_t.transpose(1, 0, 2)


# speedup vs baseline: 9.4095x; 1.0004x over previous
import functools
import math

import jax
import jax.numpy as jnp
from jax import lax
from jax.experimental import pallas as pl
from jax.experimental.pallas import tpu as pltpu
from jax.experimental.pallas import tpu_sc as plsc

NC = 2
NS = 16
NW = NC * NS
LANES = 16
NBUF = 6
PRE = 3


@jax.jit
def _sc_embed(lut, xt):
    hist, batch = xt.shape
    _, d = lut.shape
    bcols = batch // NW
    n = hist
    n_groups = (n + NBUF - 1) // NBUF
    scale = float(math.sqrt(d))
    mesh = plsc.VectorSubcoreMesh(
        core_axis_name="c", subcore_axis_name="s", num_cores=NC, num_subcores=NS
    )

    @functools.partial(
        pl.kernel,
        out_type=jax.ShapeDtypeStruct((hist, batch, d), jnp.float32),
        mesh=mesh,
        scratch_types=[
            pltpu.VMEM((hist, bcols), jnp.int32),
            pltpu.VMEM((NBUF, bcols, d), jnp.float32),
        ]
        + [pltpu.SemaphoreType.DMA] * (2 * NBUF),
        compiler_params=pltpu.CompilerParams(
            disable_bounds_checks=True, disable_semaphore_checks=True
        ),
    )
    def k(lut_hbm, idx_hbm, out_hbm, idx_v, bufs, *sems):
        gsem = sems[:NBUF]
        ssem = sems[NBUF:]
        wid = lax.axis_index("s") * NC + lax.axis_index("c")
        base = wid * bcols
        pltpu.sync_copy(idx_hbm.at[:, pl.ds(base, bcols)], idx_v)

        def gather(j, b):
            return pltpu.make_async_copy(
                lut_hbm.at[idx_v.at[j]], bufs.at[b], gsem[b]
            )

        def scatter(j, b):
            return pltpu.make_async_copy(
                bufs.at[b], out_hbm.at[j, pl.ds(base, bcols)], ssem[b]
            )

        for b in range(PRE):
            gather(b, b).start()

        def group(g, carry):
            for b in range(NBUF):
                j = g * NBUF + b
                jp = j + PRE
                bp = (b + PRE) % NBUF

                @pl.when((jp < n) & (j >= NBUF - PRE))
                def _():
                    scatter(jp - NBUF, bp).wait()
                    gather(jp, bp).start()

                @pl.when((jp < n) & (j < NBUF - PRE))
                def _():
                    gather(jp, bp).start()

                @pl.when(j < n)
                def _():
                    gather(j, b).wait()

                    @plsc.parallel_loop(0, bcols, step=1, unroll=2)
                    def _(i):
                        for t in range(d // LANES):
                            sl = pl.ds(t * LANES, LANES)
                            bufs[b, i, sl] = bufs[b, i, sl] * scale

                    scatter(j, b).start()
            return carry

        lax.fori_loop(0, n_groups, group, 0)

        for j in range(max(0, n - NBUF), n):
            scatter(j, j % NBUF).wait()

    return k(lut, xt)


def kernel(x, lut):
    out_t = _sc_embed(lut, x.T)
    return out_t.transpose(1, 0, 2)


# speedup vs baseline: 9.4475x; 1.0040x over previous
import functools
import math

import jax
import jax.numpy as jnp
from jax import lax
from jax.experimental import pallas as pl
from jax.experimental.pallas import tpu as pltpu
from jax.experimental.pallas import tpu_sc as plsc

NC = 2
NS = 16
NW = NC * NS
LANES = 16
NBUF = 6
PRE = 3


@jax.jit
def _sc_embed(lut, xt):
    hist, batch = xt.shape
    _, d = lut.shape
    bcols = batch // NW
    n = hist
    n_groups = (n + NBUF - 1) // NBUF
    scale = float(math.sqrt(d))
    mesh = plsc.VectorSubcoreMesh(
        core_axis_name="c", subcore_axis_name="s", num_cores=NC, num_subcores=NS
    )

    @functools.partial(
        pl.kernel,
        out_type=jax.ShapeDtypeStruct((hist, batch, d), jnp.float32),
        mesh=mesh,
        scratch_types=[
            pltpu.VMEM((hist, bcols), jnp.int32),
            pltpu.VMEM((NBUF, bcols, d), jnp.float32),
        ]
        + [pltpu.SemaphoreType.DMA] * (2 * NBUF),
        compiler_params=pltpu.CompilerParams(
            disable_bounds_checks=True,
            disable_semaphore_checks=True,
            skip_device_barrier=True,
        ),
    )
    def k(lut_hbm, idx_hbm, out_hbm, idx_v, bufs, *sems):
        gsem = sems[:NBUF]
        ssem = sems[NBUF:]
        wid = lax.axis_index("s") * NC + lax.axis_index("c")
        base = wid * bcols
        pltpu.sync_copy(idx_hbm.at[:, pl.ds(base, bcols)], idx_v)

        def gather(j, b):
            return pltpu.make_async_copy(
                lut_hbm.at[idx_v.at[j]], bufs.at[b], gsem[b]
            )

        def scatter(j, b):
            return pltpu.make_async_copy(
                bufs.at[b], out_hbm.at[j, pl.ds(base, bcols)], ssem[b]
            )

        for b in range(PRE):
            gather(b, b).start()

        def group(g, carry):
            for b in range(NBUF):
                j = g * NBUF + b
                jp = j + PRE
                bp = (b + PRE) % NBUF

                @pl.when((jp < n) & (j >= NBUF - PRE))
                def _():
                    scatter(jp - NBUF, bp).wait()
                    gather(jp, bp).start()

                @pl.when((jp < n) & (j < NBUF - PRE))
                def _():
                    gather(jp, bp).start()

                @pl.when(j < n)
                def _():
                    gather(j, b).wait()

                    @plsc.parallel_loop(0, bcols, step=1, unroll=2)
                    def _(i):
                        for t in range(d // LANES):
                            sl = pl.ds(t * LANES, LANES)
                            bufs[b, i, sl] = bufs[b, i, sl] * scale

                    scatter(j, b).start()
            return carry

        lax.fori_loop(0, n_groups, group, 0)

        for j in range(max(0, n - NBUF), n):
            scatter(j, j % NBUF).wait()

    return k(lut, xt)


def kernel(x, lut):
    out_t = _sc_embed(lut, x.T)
    return out_t.transpose(1, 0, 2)
